# baseline (speedup 1.0000x reference)
import functools

import jax
import jax.numpy as jnp
from jax import lax
from jax.experimental import pallas as pl
from jax.experimental.pallas import tpu as pltpu
from jax.experimental.pallas import tpu_sc as plsc

NUM_NODES = 1000000
H_DIM = 32
BATCH = 16384

_NC = 2
_NS = 16
_NW = _NC * _NS
_BPW = BATCH // _NW

_mesh = plsc.VectorSubcoreMesh(
    core_axis_name="c", subcore_axis_name="s", num_cores=_NC, num_subcores=_NS
)


@functools.partial(
    pl.kernel,
    mesh=_mesh,
    out_type=jax.ShapeDtypeStruct((BATCH, H_DIM), jnp.float32),
    scratch_types=[
        pltpu.VMEM((_BPW,), jnp.int32),
        pltpu.VMEM((_BPW, H_DIM), jnp.float32),
        pltpu.SemaphoreType.DMA,
    ],
    compiler_params=pltpu.CompilerParams(use_tc_tiling_on_sc=False),
)
def _gather_kernel(idx_hbm, table_hbm, out_hbm, idx_v, rows_v, sem):
    wid = lax.axis_index("s") * _NC + lax.axis_index("c")
    base = wid * _BPW
    pltpu.sync_copy(idx_hbm.at[pl.ds(base, _BPW)], idx_v)
    pltpu.async_copy(table_hbm.at[idx_v], rows_v, sem).wait()
    pltpu.sync_copy(rows_v, out_hbm.at[pl.ds(base, _BPW)])


def kernel(g, h, r, norm, embedding):
    idx = jnp.squeeze(h).astype(jnp.int32)
    return _gather_kernel(idx, embedding)


# speedup vs baseline: 3.9043x; 3.9043x over previous
import functools

import jax
import jax.numpy as jnp
from jax import lax
from jax.experimental import pallas as pl
from jax.experimental.pallas import tpu as pltpu
from jax.experimental.pallas import tpu_sc as plsc

NUM_NODES = 1000000
H_DIM = 32
BATCH = 16384

_NC = 2
_NS = 16
_NW = _NC * _NS
_BPW = BATCH // _NW
_C = 16
_NCHUNK = _BPW // _C
_L = 16
_TW = 128

_mesh = plsc.VectorSubcoreMesh(
    core_axis_name="c", subcore_axis_name="s", num_cores=_NC, num_subcores=_NS
)


@functools.partial(
    pl.kernel,
    mesh=_mesh,
    out_type=jax.ShapeDtypeStruct((H_DIM, BATCH), jnp.float32),
    scratch_types=[
        pltpu.VMEM((_BPW,), jnp.int32),
        pltpu.VMEM((_C, H_DIM, _TW), jnp.float32),
        pltpu.VMEM((H_DIM, _BPW), jnp.float32),
        pltpu.SemaphoreType.DMA,
    ],
    compiler_params=pltpu.CompilerParams(
        use_tc_tiling_on_sc=True, needs_layout_passes=False
    ),
)
def _gather_kernel(idx_hbm, table_hbm, out_hbm, idx_v, blocks_v, cols_v, sem):
    wid = lax.axis_index("s") * _NC + lax.axis_index("c")
    base = wid * _BPW
    pltpu.sync_copy(idx_hbm.at[pl.ds(base, _BPW)], idx_v)

    def chunk_body(c):
        cbase = c * _C
        for g in range(_C // _L):
            starts = idx_v[pl.ds(cbase + g * _L, _L)] & jnp.int32(-_TW)
            for i in range(_L):
                start = pl.multiple_of(starts[i], _TW)
                pltpu.async_copy(
                    table_hbm.at[:, pl.ds(start, _TW)],
                    blocks_v.at[g * _L + i],
                    sem,
                )
        for i in range(_C):
            pltpu.make_async_copy(
                table_hbm.at[:, pl.ds(0, _TW)], blocks_v.at[i], sem
            ).wait()
        for g in range(_C // _L):
            lvec = idx_v[pl.ds(cbase + g * _L, _L)] & jnp.int32(_TW - 1)
            bvec = lax.iota(jnp.int32, _L) + jnp.int32(g * _L)
            for j in range(H_DIM):
                jvec = jnp.full((_L,), j, jnp.int32)
                vals = plsc.load_gather(blocks_v, [bvec, jvec, lvec])
                cols_v[j, pl.ds(cbase + g * _L, _L)] = vals

    pl.loop(0, _NCHUNK)(chunk_body)
    pltpu.sync_copy(cols_v, out_hbm.at[:, pl.ds(base, _BPW)])


def kernel(g, h, r, norm, embedding):
    idx = jnp.squeeze(h).astype(jnp.int32)
    out_t = _gather_kernel(idx, embedding.T)
    return out_t.T
